# baseline (speedup 1.0000x reference)
import functools
import jax
import jax.numpy as jnp
from jax import lax
from jax.experimental import pallas as pl
from jax.experimental.pallas import tpu as pltpu
from jax.experimental.pallas import tpu_sc as plsc

N = 50000
E = 800000
H = 64
N_LAYERS = 6
BN = 1000
BE = 1024

NHALF = 25000
BATCH = 32
NLISTS = 128
PAD_E = 804864
ACC_ROWS = 12672
DUMP = 12544
ZCH = 792
DCH = 784
OUT_ROWS = 25008

_f32 = jnp.float32


def _full(shape):
    return pl.BlockSpec(shape, lambda *_: tuple(0 for _ in shape))


def _rows(shape):
    return pl.BlockSpec(shape, lambda i: (0,) * (len(shape) - 2) + (i, 0))


def _enc_body(x_ref, w1, b1, w2, b2, a0, b0, h_ref, p_ref, q_ref):
    t = jnp.maximum(jnp.dot(x_ref[...], w1[...], preferred_element_type=_f32, precision=lax.Precision.HIGHEST) + b1[...], 0.0)
    h = jnp.dot(t, w2[...], preferred_element_type=_f32, precision=lax.Precision.HIGHEST) + b2[...]
    h_ref[...] = h
    p_ref[...] = jnp.dot(h, a0[...], preferred_element_type=_f32, precision=lax.Precision.HIGHEST)
    q_ref[...] = jnp.dot(h, b0[...], preferred_element_type=_f32, precision=lax.Precision.HIGHEST)


def _enc_nodes(x, w1, b1, w2, b2, a0, b0):
    return pl.pallas_call(
        _enc_body,
        grid=(N // BN,),
        in_specs=[
            pl.BlockSpec((BN, x.shape[1]), lambda i: (i, 0)),
            _full(w1.shape), _full(b1.shape), _full(w2.shape), _full(b2.shape),
            _full(a0.shape), _full(b0.shape),
        ],
        out_specs=[_rows((BN, H)), _rows((BN, H)), _rows((BN, H))],
        out_shape=[jax.ShapeDtypeStruct((N, H), _f32)] * 3,
    )(x, w1, b1, w2, b2, a0, b0)


def _ec_body(ea_ref, w1, b1, w2, b2, cs, b1s, ec_ref):
    t = jnp.maximum(jnp.dot(ea_ref[...], w1[...], preferred_element_type=_f32, precision=lax.Precision.HIGHEST) + b1[...], 0.0)
    e = jnp.dot(t, w2[...], preferred_element_type=_f32, precision=lax.Precision.HIGHEST) + b2[...]
    for l in range(N_LAYERS):
        ec_ref[l] = jnp.dot(e, cs[l], preferred_element_type=_f32, precision=lax.Precision.HIGHEST) + b1s[l]


def _ec_all(ea, w1, b1, w2, b2, cs, b1s):
    return pl.pallas_call(
        _ec_body,
        grid=(PAD_E // BE,),
        in_specs=[
            pl.BlockSpec((BE, ea.shape[1]), lambda i: (i, 0)),
            _full(w1.shape), _full(b1.shape), _full(w2.shape), _full(b2.shape),
            pl.BlockSpec(cs.shape, lambda i: (0, 0, 0)),
            pl.BlockSpec(b1s.shape, lambda i: (0, 0, 0)),
        ],
        out_specs=pl.BlockSpec((N_LAYERS, BE, H), lambda i: (0, i, 0)),
        out_shape=jax.ShapeDtypeStruct((N_LAYERS, PAD_E, H), _f32),
    )(ea, w1, b1, w2, b2, cs, b1s)


def _upd_body(h_ref, s_ref, deg_ref, w2e, b2e, v1a, v1b, b1n, v2, b2n, an, bn,
              h_out, p_out, q_out):
    h = h_ref[...]
    agg = jnp.dot(s_ref[...], w2e[...], preferred_element_type=_f32, precision=lax.Precision.HIGHEST) + deg_ref[...] * b2e[...]
    t = jnp.maximum(
        jnp.dot(h, v1a[...], preferred_element_type=_f32, precision=lax.Precision.HIGHEST)
        + jnp.dot(agg, v1b[...], preferred_element_type=_f32, precision=lax.Precision.HIGHEST) + b1n[...], 0.0)
    hn = h + jnp.dot(t, v2[...], preferred_element_type=_f32, precision=lax.Precision.HIGHEST) + b2n[...]
    h_out[...] = hn
    p_out[...] = jnp.dot(hn, an[...], preferred_element_type=_f32, precision=lax.Precision.HIGHEST)
    q_out[...] = jnp.dot(hn, bn[...], preferred_element_type=_f32, precision=lax.Precision.HIGHEST)


def _node_update(h, s, deg, w2e, b2e, v1a, v1b, b1n, v2, b2n, an, bn):
    return pl.pallas_call(
        _upd_body,
        grid=(N // BN,),
        in_specs=[
            _rows((BN, H)), _rows((BN, H)), pl.BlockSpec((BN, 1), lambda i: (i, 0)),
            _full(w2e.shape), _full(b2e.shape), _full(v1a.shape), _full(v1b.shape),
            _full(b1n.shape), _full(v2.shape), _full(b2n.shape),
            _full(an.shape), _full(bn.shape),
        ],
        out_specs=[_rows((BN, H))] * 3,
        out_shape=[jax.ShapeDtypeStruct((N, H), _f32)] * 3,
    )(h, s, deg, w2e, b2e, v1a, v1b, b1n, v2, b2n, an, bn)


def _last_body(h_ref, s_ref, deg_ref, w2e, b2e, v1a, v1b, b1n, v2, b2n,
               d1, db1, d2, db2, d3, db3, fac_ref, out_ref):
    h = h_ref[...]
    agg = jnp.dot(s_ref[...], w2e[...], preferred_element_type=_f32, precision=lax.Precision.HIGHEST) + deg_ref[...] * b2e[...]
    t = jnp.maximum(
        jnp.dot(h, v1a[...], preferred_element_type=_f32, precision=lax.Precision.HIGHEST)
        + jnp.dot(agg, v1b[...], preferred_element_type=_f32, precision=lax.Precision.HIGHEST) + b1n[...], 0.0)
    hn = h + jnp.dot(t, v2[...], preferred_element_type=_f32, precision=lax.Precision.HIGHEST) + b2n[...]
    u = jnp.maximum(jnp.dot(hn, d1[...], preferred_element_type=_f32, precision=lax.Precision.HIGHEST) + db1[...], 0.0)
    u = jnp.maximum(jnp.dot(u, d2[...], preferred_element_type=_f32, precision=lax.Precision.HIGHEST) + db2[...], 0.0)
    raw = jnp.dot(u, d3[...], preferred_element_type=_f32, precision=lax.Precision.HIGHEST) + db3[...]
    out_ref[...] = raw * fac_ref[...]


def _last_update(h, s, deg, w2e, b2e, v1a, v1b, b1n, v2, b2n, dec_ws, fac):
    d1, db1, d2, db2, d3, db3 = dec_ws
    return pl.pallas_call(
        _last_body,
        grid=(N // BN,),
        in_specs=[
            _rows((BN, H)), _rows((BN, H)), pl.BlockSpec((BN, 1), lambda i: (i, 0)),
            _full(w2e.shape), _full(b2e.shape), _full(v1a.shape), _full(v1b.shape),
            _full(b1n.shape), _full(v2.shape), _full(b2n.shape),
            _full(d1.shape), _full(db1.shape), _full(d2.shape), _full(db2.shape),
            _full(d3.shape), _full(db3.shape),
            pl.BlockSpec((BN, 3), lambda i: (i, 0)),
        ],
        out_specs=pl.BlockSpec((BN, 3), lambda i: (i, 0)),
        out_shape=jax.ShapeDtypeStruct((N, 3), _f32),
    )(h, s, deg, w2e, b2e, v1a, v1b, b1n, v2, b2n, d1, db1, d2, db2, d3, db3, fac)


def _build_lists(src, dst, edge_attr):
    eid = jnp.arange(E, dtype=jnp.int32)
    p_of = eid // NHALF
    c_of = (dst >= NHALF).astype(jnp.int32)
    key = p_of * 4 + c_of * 2 + (dst & 1)
    order = jnp.argsort(key, stable=True)
    cnts = jnp.bincount(key, length=NLISTS).astype(jnp.int32)
    nb = ((cnts + BATCH - 1) // BATCH).astype(jnp.int32)
    padded = nb * BATCH
    astarts = (jnp.cumsum(padded) - padded).astype(jnp.int32)
    gstarts = (jnp.cumsum(cnts) - cnts).astype(jnp.int32)
    slots = jnp.arange(PAD_E, dtype=jnp.int32)
    g = (jnp.searchsorted(astarts, slots, side='right') - 1).astype(jnp.int32)
    g = jnp.clip(g, 0, NLISTS - 1)
    rank = slots - astarts[g]
    valid = rank < cnts[g]
    src_idx = jnp.clip(gstarts[g] + jnp.minimum(rank, cnts[g] - 1), 0, E - 1)
    e_of = order[src_idx]
    dg = dst[e_of]
    co = c_of[e_of]
    lsrc = jnp.where(valid, src[e_of], 0)
    ldstg = jnp.where(valid, dg, 0)
    ldstl = jnp.where(valid, (dg - co * NHALF) >> 1, DUMP)
    ea_s = jnp.where(valid[:, None], edge_attr[e_of], 0.0)
    counts = jnp.broadcast_to(nb[:, None], (NLISTS, 16)).reshape(-1)
    starts = jnp.broadcast_to(astarts[:, None], (NLISTS, 16)).reshape(-1)
    return lsrc, ldstg, ldstl, counts, starts, ea_s


def _sc_mesh():
    return plsc.VectorSubcoreMesh(core_axis_name="c", subcore_axis_name="s")


def _zero_buf(buf, rows, width):
    zrow = jnp.zeros((16,), _f32)

    def zb(j, _):
        for kk in range(width // 16):
            buf[j, pl.ds(kk * 16, 16)] = zrow
        return 0
    lax.fori_loop(0, rows, zb, 0)


def _zero_acc(acc, zb, s):
    base = s * ZCH

    def za(j, _):
        pltpu.sync_copy(zb, acc.at[pl.ds(base + j * 32, 32)])
        return 0
    lax.fori_loop(0, 24, za, 0)
    pltpu.sync_copy(zb.at[pl.ds(0, 24)], acc.at[pl.ds(base + 768, 24)])


def _drain_acc(acc, out_h, c, s):
    row0 = s * DCH
    gbase = c * 12504 + row0

    def dr(j, _):
        pltpu.sync_copy(acc.at[pl.ds(row0 + j * 128, 128)],
                        out_h.at[pl.ds(gbase + j * 128, 128)])
        return 0
    lax.fori_loop(0, 5, dr, 0)

    @pl.when(s < 15)
    def _t1():
        pltpu.sync_copy(acc.at[pl.ds(row0 + 640, 128)],
                        out_h.at[pl.ds(gbase + 640, 128)])
        pltpu.sync_copy(acc.at[pl.ds(row0 + 768, 16)],
                        out_h.at[pl.ds(gbase + 768, 16)])

    @pl.when(s == 15)
    def _t2():
        pltpu.sync_copy(acc.at[pl.ds(row0 + 640, 104)],
                        out_h.at[pl.ds(gbase + 640, 104)])


def _sc_layer(PQ, ecl, lsrc, ldstg, ldstl, counts, starts):
    @functools.partial(
        pl.kernel, mesh=_sc_mesh(),
        out_type=jax.ShapeDtypeStruct((OUT_ROWS, 2 * H), _f32),
        scratch_types=[
            pltpu.VMEM_SHARED((ACC_ROWS, 2 * H), _f32),
            pltpu.VMEM((BATCH,), jnp.int32),
            pltpu.VMEM((BATCH,), jnp.int32),
            pltpu.VMEM((BATCH,), jnp.int32),
            pltpu.VMEM((BATCH, 2 * H), _f32),
            pltpu.VMEM((BATCH, 2 * H), _f32),
            pltpu.VMEM((BATCH, H), _f32),
            pltpu.VMEM((BATCH, 2 * H), _f32),
            pltpu.VMEM((16,), jnp.int32),
            pltpu.VMEM((16,), jnp.int32),
            pltpu.SemaphoreType.DMA,
            pltpu.SemaphoreType.DMA,
            pltpu.SemaphoreType.DMA,
        ],
    )
    def k(PQ_h, ec_h, lsrc_h, ldstg_h, ldstl_h, cnt_h, st_h, out_h,
          acc, srcv, dstgv, dstlv, bufS, bufD, bufE, bufZ, cntv, stv,
          sem1, sem2, sem3):
        c = lax.axis_index("c")
        s = lax.axis_index("s")
        _zero_buf(bufZ, BATCH, 2 * H)
        _zero_acc(acc, bufZ, s)
        plsc.subcore_barrier()

        def do_list(p, par):
            g = p * 4 + c * 2 + par
            pltpu.sync_copy(cnt_h.at[pl.ds(g * 16, 16)], cntv)
            pltpu.sync_copy(st_h.at[pl.ds(g * 16, 16)], stv)
            nb = cntv[pl.ds(0, 16)][0]
            st = pl.multiple_of(stv[pl.ds(0, 16)][0], BATCH)

            def body(i, _):
                off = pl.multiple_of(st + i * BATCH, BATCH)
                pltpu.sync_copy(lsrc_h.at[pl.ds(off, BATCH)], srcv)
                pltpu.sync_copy(ldstg_h.at[pl.ds(off, BATCH)], dstgv)
                pltpu.sync_copy(ldstl_h.at[pl.ds(off, BATCH)], dstlv)
                cp1 = pltpu.async_copy(PQ_h.at[srcv], bufS, sem1)
                cp2 = pltpu.async_copy(PQ_h.at[dstgv], bufD, sem2)
                cp3 = pltpu.async_copy(ec_h.at[pl.ds(off, BATCH)], bufE, sem3)
                cp1.wait()
                cp2.wait()
                cp3.wait()

                def rowf(j, _):
                    for kk in range(4):
                        bufZ[j, pl.ds(par * H + kk * 16, 16)] = jnp.maximum(
                            bufS[j, pl.ds(kk * 16, 16)]
                            + bufD[j, pl.ds(H + kk * 16, 16)]
                            + bufE[j, pl.ds(kk * 16, 16)], 0.0)
                    return 0
                lax.fori_loop(0, BATCH, rowf, 0)
                pltpu.sync_copy(bufZ, acc.at[dstlv], add=True)
                return 0
            lax.fori_loop(0, nb, body, 0)

        for par in (0, 1):
            zrow = jnp.zeros((16,), _f32)

            def zhalf(j, _):
                for kk in range(4):
                    bufZ[j, pl.ds((1 - par) * H + kk * 16, 16)] = zrow
                return 0
            lax.fori_loop(0, BATCH, zhalf, 0)
            do_list(2 * s, par)
            do_list(2 * s + 1, par)
        plsc.subcore_barrier()
        _drain_acc(acc, out_h, c, s)

    return k(PQ, ecl, lsrc, ldstg, ldstl, counts, starts)


def _unpack_s(out):
    return jnp.concatenate([out[0:12500], out[12504:25004]], axis=0).reshape(N, H)


def kernel(x, edge_index, edge_attr, u_c, theta_c, bc_disp, bc_rot, params):
    src = edge_index[0]
    dst = edge_index[1]

    def r2(b):
        return b.reshape(1, -1)

    ne = params['node_enc']
    ee = params['edge_enc']
    mp = params['mp']
    dec = params['dec']

    As = [lp['edge'][0][0][0:H] for lp in mp]
    Bs = [lp['edge'][0][0][H:2 * H] for lp in mp]
    Cs = jnp.stack([lp['edge'][0][0][2 * H:3 * H] for lp in mp])
    b1s = jnp.stack([lp['edge'][0][1].reshape(1, H) for lp in mp])
    V1as = [lp['node'][0][0][0:H] for lp in mp]
    V1bs = [lp['node'][0][0][H:2 * H] for lp in mp]

    lsrc, ldstg, ldstl, counts, starts, ea_s = _build_lists(src, dst, edge_attr)

    h, P, Q = _enc_nodes(x, ne[0][0], r2(ne[0][1]), ne[1][0], r2(ne[1][1]),
                         As[0], Bs[0])
    ec = _ec_all(ea_s, ee[0][0], r2(ee[0][1]), ee[1][0], r2(ee[1][1]),
                 Cs, b1s)

    deg = _unpack_s(_sc_layer(jnp.zeros((N, 2 * H), _f32),
                              jnp.ones((PAD_E, H), _f32),
                              lsrc, ldstg, ldstl, counts, starts))[:, 0:1]

    fac = jnp.concatenate([
        u_c.reshape(N, 1) * (1.0 - bc_disp),
        u_c.reshape(N, 1) * (1.0 - bc_disp),
        theta_c.reshape(N, 1) * (1.0 - bc_rot)], axis=1)

    for l in range(N_LAYERS):
        lp = mp[l]
        s = _unpack_s(_sc_layer(jnp.concatenate([P, Q], axis=1), ec[l],
                                lsrc, ldstg, ldstl, counts, starts))
        w2e, b2e = lp['edge'][1][0], r2(lp['edge'][1][1])
        b1n = r2(lp['node'][0][1])
        v2, b2n = lp['node'][1][0], r2(lp['node'][1][1])
        if l < N_LAYERS - 1:
            h, P, Q = _node_update(h, s, deg, w2e, b2e, V1as[l], V1bs[l], b1n,
                                   v2, b2n, As[l + 1], Bs[l + 1])
        else:
            dec_ws = (dec[0][0], r2(dec[0][1]), dec[1][0], r2(dec[1][1]),
                      dec[2][0], r2(dec[2][1]))
            out = _last_update(h, s, deg, w2e, b2e, V1as[l], V1bs[l], b1n,
                               v2, b2n, dec_ws, fac)
    return out


# speedup vs baseline: 2.9619x; 2.9619x over previous
import functools
import jax
import jax.numpy as jnp
from jax import lax
from jax.experimental import pallas as pl
from jax.experimental.pallas import tpu as pltpu
from jax.experimental.pallas import tpu_sc as plsc

N = 50000
E = 800000
H = 64
N_LAYERS = 6
BN = 1000
BE = 1024

NHALF = 25000
BATCH = 32
NLISTS = 4
PAD_E = 802816
ACC_ROWS = 12672
DUMP = 12544
ZCH = 792
DCH = 784
OUT_ROWS = 25008

_f32 = jnp.float32


def _full(shape):
    return pl.BlockSpec(shape, lambda *_: tuple(0 for _ in shape))


def _rows(shape):
    return pl.BlockSpec(shape, lambda i: (0,) * (len(shape) - 2) + (i, 0))


def _enc_body(x_ref, w1, b1, w2, b2, a0, b0, h_ref, p_ref, q_ref):
    t = jnp.maximum(jnp.dot(x_ref[...], w1[...], preferred_element_type=_f32, precision=lax.Precision.HIGHEST) + b1[...], 0.0)
    h = jnp.dot(t, w2[...], preferred_element_type=_f32, precision=lax.Precision.HIGHEST) + b2[...]
    h_ref[...] = h
    p_ref[...] = jnp.dot(h, a0[...], preferred_element_type=_f32, precision=lax.Precision.HIGHEST)
    q_ref[...] = jnp.dot(h, b0[...], preferred_element_type=_f32, precision=lax.Precision.HIGHEST)


def _enc_nodes(x, w1, b1, w2, b2, a0, b0):
    return pl.pallas_call(
        _enc_body,
        grid=(N // BN,),
        in_specs=[
            pl.BlockSpec((BN, x.shape[1]), lambda i: (i, 0)),
            _full(w1.shape), _full(b1.shape), _full(w2.shape), _full(b2.shape),
            _full(a0.shape), _full(b0.shape),
        ],
        out_specs=[_rows((BN, H)), _rows((BN, H)), _rows((BN, H))],
        out_shape=[jax.ShapeDtypeStruct((N, H), _f32)] * 3,
    )(x, w1, b1, w2, b2, a0, b0)


def _ec_body(ea_ref, w1, b1, w2, b2, cs, b1s, ec_ref):
    t = jnp.maximum(jnp.dot(ea_ref[...], w1[...], preferred_element_type=_f32, precision=lax.Precision.HIGHEST) + b1[...], 0.0)
    e = jnp.dot(t, w2[...], preferred_element_type=_f32, precision=lax.Precision.HIGHEST) + b2[...]
    for l in range(N_LAYERS):
        ec_ref[l] = jnp.dot(e, cs[l], preferred_element_type=_f32, precision=lax.Precision.HIGHEST) + b1s[l]


def _ec_all(ea, w1, b1, w2, b2, cs, b1s):
    return pl.pallas_call(
        _ec_body,
        grid=(PAD_E // BE,),
        in_specs=[
            pl.BlockSpec((BE, ea.shape[1]), lambda i: (i, 0)),
            _full(w1.shape), _full(b1.shape), _full(w2.shape), _full(b2.shape),
            pl.BlockSpec(cs.shape, lambda i: (0, 0, 0)),
            pl.BlockSpec(b1s.shape, lambda i: (0, 0, 0)),
        ],
        out_specs=pl.BlockSpec((N_LAYERS, BE, H), lambda i: (0, i, 0)),
        out_shape=jax.ShapeDtypeStruct((N_LAYERS, PAD_E, H), _f32),
    )(ea, w1, b1, w2, b2, cs, b1s)


def _upd_body(h_ref, s_ref, deg_ref, w2e, b2e, v1a, v1b, b1n, v2, b2n, an, bn,
              h_out, p_out, q_out):
    h = h_ref[...]
    agg = jnp.dot(s_ref[...], w2e[...], preferred_element_type=_f32, precision=lax.Precision.HIGHEST) + deg_ref[...] * b2e[...]
    t = jnp.maximum(
        jnp.dot(h, v1a[...], preferred_element_type=_f32, precision=lax.Precision.HIGHEST)
        + jnp.dot(agg, v1b[...], preferred_element_type=_f32, precision=lax.Precision.HIGHEST) + b1n[...], 0.0)
    hn = h + jnp.dot(t, v2[...], preferred_element_type=_f32, precision=lax.Precision.HIGHEST) + b2n[...]
    h_out[...] = hn
    p_out[...] = jnp.dot(hn, an[...], preferred_element_type=_f32, precision=lax.Precision.HIGHEST)
    q_out[...] = jnp.dot(hn, bn[...], preferred_element_type=_f32, precision=lax.Precision.HIGHEST)


def _node_update(h, s, deg, w2e, b2e, v1a, v1b, b1n, v2, b2n, an, bn):
    return pl.pallas_call(
        _upd_body,
        grid=(N // BN,),
        in_specs=[
            _rows((BN, H)), _rows((BN, H)), pl.BlockSpec((BN, 1), lambda i: (i, 0)),
            _full(w2e.shape), _full(b2e.shape), _full(v1a.shape), _full(v1b.shape),
            _full(b1n.shape), _full(v2.shape), _full(b2n.shape),
            _full(an.shape), _full(bn.shape),
        ],
        out_specs=[_rows((BN, H))] * 3,
        out_shape=[jax.ShapeDtypeStruct((N, H), _f32)] * 3,
    )(h, s, deg, w2e, b2e, v1a, v1b, b1n, v2, b2n, an, bn)


def _last_body(h_ref, s_ref, deg_ref, w2e, b2e, v1a, v1b, b1n, v2, b2n,
               d1, db1, d2, db2, d3, db3, fac_ref, out_ref):
    h = h_ref[...]
    agg = jnp.dot(s_ref[...], w2e[...], preferred_element_type=_f32, precision=lax.Precision.HIGHEST) + deg_ref[...] * b2e[...]
    t = jnp.maximum(
        jnp.dot(h, v1a[...], preferred_element_type=_f32, precision=lax.Precision.HIGHEST)
        + jnp.dot(agg, v1b[...], preferred_element_type=_f32, precision=lax.Precision.HIGHEST) + b1n[...], 0.0)
    hn = h + jnp.dot(t, v2[...], preferred_element_type=_f32, precision=lax.Precision.HIGHEST) + b2n[...]
    u = jnp.maximum(jnp.dot(hn, d1[...], preferred_element_type=_f32, precision=lax.Precision.HIGHEST) + db1[...], 0.0)
    u = jnp.maximum(jnp.dot(u, d2[...], preferred_element_type=_f32, precision=lax.Precision.HIGHEST) + db2[...], 0.0)
    raw = jnp.dot(u, d3[...], preferred_element_type=_f32, precision=lax.Precision.HIGHEST) + db3[...]
    out_ref[...] = raw * fac_ref[...]


def _last_update(h, s, deg, w2e, b2e, v1a, v1b, b1n, v2, b2n, dec_ws, fac):
    d1, db1, d2, db2, d3, db3 = dec_ws
    return pl.pallas_call(
        _last_body,
        grid=(N // BN,),
        in_specs=[
            _rows((BN, H)), _rows((BN, H)), pl.BlockSpec((BN, 1), lambda i: (i, 0)),
            _full(w2e.shape), _full(b2e.shape), _full(v1a.shape), _full(v1b.shape),
            _full(b1n.shape), _full(v2.shape), _full(b2n.shape),
            _full(d1.shape), _full(db1.shape), _full(d2.shape), _full(db2.shape),
            _full(d3.shape), _full(db3.shape),
            pl.BlockSpec((BN, 3), lambda i: (i, 0)),
        ],
        out_specs=pl.BlockSpec((BN, 3), lambda i: (i, 0)),
        out_shape=jax.ShapeDtypeStruct((N, 3), _f32),
    )(h, s, deg, w2e, b2e, v1a, v1b, b1n, v2, b2n, d1, db1, d2, db2, d3, db3, fac)


def _build_lists(src, dst, edge_attr):
    c_of = (dst >= NHALF).astype(jnp.int32)
    key = c_of * 2 + (dst & 1)
    pos = jnp.zeros((E,), jnp.int32)
    cnts = []
    for g in range(NLISTS):
        m = (key == g)
        pos = jnp.where(m, jnp.cumsum(m.astype(jnp.int32)) - 1, pos)
        cnts.append(jnp.sum(m.astype(jnp.int32)))
    cnts = jnp.stack(cnts)
    totb = (cnts + BATCH - 1) // BATCH
    subb = (totb + 15) // 16
    padded = subb * 16 * BATCH
    astarts = jnp.cumsum(padded) - padded
    astart_e = jnp.zeros((E,), jnp.int32)
    for g in range(NLISTS):
        astart_e = jnp.where(key == g, astarts[g], astart_e)
    slot = astart_e + pos
    row = (dst - c_of * NHALF) >> 1
    lsrc = jnp.zeros((PAD_E,), jnp.int32).at[slot].set(src)
    ldstg = jnp.zeros((PAD_E,), jnp.int32).at[slot].set(dst)
    ldstl = jnp.full((PAD_E,), DUMP, jnp.int32).at[slot].set(row)
    ea_s = jnp.zeros((PAD_E, edge_attr.shape[1]), _f32).at[slot].set(edge_attr)
    def splat(v):
        return jnp.broadcast_to(v.astype(jnp.int32)[:, None], (NLISTS, 16)).reshape(-1)
    return lsrc, ldstg, ldstl, splat(subb), splat(totb), splat(astarts), ea_s


def _sc_mesh():
    return plsc.VectorSubcoreMesh(core_axis_name="c", subcore_axis_name="s")


def _zero_buf(buf, rows, width):
    zrow = jnp.zeros((16,), _f32)

    def zb(j, _):
        for kk in range(width // 16):
            buf[j, pl.ds(kk * 16, 16)] = zrow
        return 0
    lax.fori_loop(0, rows, zb, 0)


def _zero_acc(acc, zb, s):
    base = s * ZCH

    def za(j, _):
        pltpu.sync_copy(zb, acc.at[pl.ds(base + j * 32, 32)])
        return 0
    lax.fori_loop(0, 24, za, 0)
    pltpu.sync_copy(zb.at[pl.ds(0, 24)], acc.at[pl.ds(base + 768, 24)])


def _drain_acc(acc, out_h, c, s):
    row0 = s * DCH
    gbase = c * 12504 + row0

    def dr(j, _):
        pltpu.sync_copy(acc.at[pl.ds(row0 + j * 128, 128)],
                        out_h.at[pl.ds(gbase + j * 128, 128)])
        return 0
    lax.fori_loop(0, 5, dr, 0)

    @pl.when(s < 15)
    def _t1():
        pltpu.sync_copy(acc.at[pl.ds(row0 + 640, 128)],
                        out_h.at[pl.ds(gbase + 640, 128)])
        pltpu.sync_copy(acc.at[pl.ds(row0 + 768, 16)],
                        out_h.at[pl.ds(gbase + 768, 16)])

    @pl.when(s == 15)
    def _t2():
        pltpu.sync_copy(acc.at[pl.ds(row0 + 640, 104)],
                        out_h.at[pl.ds(gbase + 640, 104)])


def _sc_layer(PQ, ecl, lsrc, ldstg, ldstl, subb, totb, starts):
    @functools.partial(
        pl.kernel, mesh=_sc_mesh(),
        out_type=jax.ShapeDtypeStruct((OUT_ROWS, 2 * H), _f32),
        scratch_types=[
            pltpu.VMEM_SHARED((ACC_ROWS, 2 * H), _f32),
            pltpu.VMEM((BATCH,), jnp.int32),
            pltpu.VMEM((BATCH,), jnp.int32),
            pltpu.VMEM((BATCH,), jnp.int32),
            pltpu.VMEM((BATCH, 2 * H), _f32),
            pltpu.VMEM((BATCH, 2 * H), _f32),
            pltpu.VMEM((BATCH, H), _f32),
            pltpu.VMEM((BATCH, 2 * H), _f32),
            pltpu.VMEM((16,), jnp.int32),
            pltpu.VMEM((16,), jnp.int32),
            pltpu.VMEM((16,), jnp.int32),
            pltpu.SemaphoreType.DMA,
            pltpu.SemaphoreType.DMA,
            pltpu.SemaphoreType.DMA,
        ],
    )
    def k(PQ_h, ec_h, lsrc_h, ldstg_h, ldstl_h, sb_h, tb_h, st_h, out_h,
          acc, srcv, dstgv, dstlv, bufS, bufD, bufE, bufZ, cntv, stv, tbv,
          sem1, sem2, sem3):
        c = lax.axis_index("c")
        s = lax.axis_index("s")
        _zero_buf(bufZ, BATCH, 2 * H)
        _zero_acc(acc, bufZ, s)
        plsc.subcore_barrier()

        def do_list(par):
            g = c * 2 + par
            pltpu.sync_copy(sb_h.at[pl.ds(g * 16, 16)], cntv)
            pltpu.sync_copy(tb_h.at[pl.ds(g * 16, 16)], tbv)
            pltpu.sync_copy(st_h.at[pl.ds(g * 16, 16)], stv)
            sb = cntv[pl.ds(0, 16)][0]
            tb = tbv[pl.ds(0, 16)][0]
            nb = jnp.minimum(sb, jnp.maximum(tb - s * sb, 0))
            st = pl.multiple_of(stv[pl.ds(0, 16)][0] + s * sb * BATCH, BATCH)

            def body(i, _):
                off = pl.multiple_of(st + i * BATCH, BATCH)
                pltpu.sync_copy(lsrc_h.at[pl.ds(off, BATCH)], srcv)
                pltpu.sync_copy(ldstg_h.at[pl.ds(off, BATCH)], dstgv)
                pltpu.sync_copy(ldstl_h.at[pl.ds(off, BATCH)], dstlv)
                cp1 = pltpu.async_copy(PQ_h.at[srcv], bufS, sem1)
                cp2 = pltpu.async_copy(PQ_h.at[dstgv], bufD, sem2)
                cp3 = pltpu.async_copy(ec_h.at[pl.ds(off, BATCH)], bufE, sem3)
                cp1.wait()
                cp2.wait()
                cp3.wait()

                def rowf(j, _):
                    for kk in range(4):
                        bufZ[j, pl.ds(par * H + kk * 16, 16)] = jnp.maximum(
                            bufS[j, pl.ds(kk * 16, 16)]
                            + bufD[j, pl.ds(H + kk * 16, 16)]
                            + bufE[j, pl.ds(kk * 16, 16)], 0.0)
                    return 0
                lax.fori_loop(0, BATCH, rowf, 0)
                pltpu.sync_copy(bufZ, acc.at[dstlv], add=True)
                return 0
            lax.fori_loop(0, nb, body, 0)

        for par in (0, 1):
            zrow = jnp.zeros((16,), _f32)

            def zhalf(j, _):
                for kk in range(4):
                    bufZ[j, pl.ds((1 - par) * H + kk * 16, 16)] = zrow
                return 0
            lax.fori_loop(0, BATCH, zhalf, 0)
            do_list(par)
        plsc.subcore_barrier()
        _drain_acc(acc, out_h, c, s)

    return k(PQ, ecl, lsrc, ldstg, ldstl, subb, totb, starts)


def _unpack_s(out):
    return jnp.concatenate([out[0:12500], out[12504:25004]], axis=0).reshape(N, H)


def kernel(x, edge_index, edge_attr, u_c, theta_c, bc_disp, bc_rot, params):
    src = edge_index[0]
    dst = edge_index[1]

    def r2(b):
        return b.reshape(1, -1)

    ne = params['node_enc']
    ee = params['edge_enc']
    mp = params['mp']
    dec = params['dec']

    As = [lp['edge'][0][0][0:H] for lp in mp]
    Bs = [lp['edge'][0][0][H:2 * H] for lp in mp]
    Cs = jnp.stack([lp['edge'][0][0][2 * H:3 * H] for lp in mp])
    b1s = jnp.stack([lp['edge'][0][1].reshape(1, H) for lp in mp])
    V1as = [lp['node'][0][0][0:H] for lp in mp]
    V1bs = [lp['node'][0][0][H:2 * H] for lp in mp]

    lsrc, ldstg, ldstl, subb, totb, starts, ea_s = _build_lists(src, dst, edge_attr)

    h, P, Q = _enc_nodes(x, ne[0][0], r2(ne[0][1]), ne[1][0], r2(ne[1][1]),
                         As[0], Bs[0])
    ec = _ec_all(ea_s, ee[0][0], r2(ee[0][1]), ee[1][0], r2(ee[1][1]),
                 Cs, b1s)

    deg = _unpack_s(_sc_layer(jnp.zeros((N, 2 * H), _f32),
                              jnp.ones((PAD_E, H), _f32),
                              lsrc, ldstg, ldstl, subb, totb, starts))[:, 0:1]

    fac = jnp.concatenate([
        u_c.reshape(N, 1) * (1.0 - bc_disp),
        u_c.reshape(N, 1) * (1.0 - bc_disp),
        theta_c.reshape(N, 1) * (1.0 - bc_rot)], axis=1)

    for l in range(N_LAYERS):
        lp = mp[l]
        s = _unpack_s(_sc_layer(jnp.concatenate([P, Q], axis=1), ec[l],
                                lsrc, ldstg, ldstl, subb, totb, starts))
        w2e, b2e = lp['edge'][1][0], r2(lp['edge'][1][1])
        b1n = r2(lp['node'][0][1])
        v2, b2n = lp['node'][1][0], r2(lp['node'][1][1])
        if l < N_LAYERS - 1:
            h, P, Q = _node_update(h, s, deg, w2e, b2e, V1as[l], V1bs[l], b1n,
                                   v2, b2n, As[l + 1], Bs[l + 1])
        else:
            dec_ws = (dec[0][0], r2(dec[0][1]), dec[1][0], r2(dec[1][1]),
                      dec[2][0], r2(dec[2][1]))
            out = _last_update(h, s, deg, w2e, b2e, V1as[l], V1bs[l], b1n,
                               v2, b2n, dec_ws, fac)
    return out


# speedup vs baseline: 3.1761x; 1.0723x over previous
import functools
import jax
import jax.numpy as jnp
from jax import lax
from jax.experimental import pallas as pl
from jax.experimental.pallas import tpu as pltpu
from jax.experimental.pallas import tpu_sc as plsc

N = 50000
E = 800000
H = 64
N_LAYERS = 6
BN = 1000
BE = 1024

NHALF = 25000
BATCH = 48
NLISTS = 4
PAD_E = 804864
ACC_ROWS = 12672
DUMP = 12544
ZCH = 792
DCH = 784
OUT_ROWS = 25008

_f32 = jnp.float32


def _full(shape):
    return pl.BlockSpec(shape, lambda *_: tuple(0 for _ in shape))


def _rows(shape):
    return pl.BlockSpec(shape, lambda i: (0,) * (len(shape) - 2) + (i, 0))


def _enc_body(x_ref, w1, b1, w2, b2, a0, b0, h_ref, p_ref, q_ref):
    t = jnp.maximum(jnp.dot(x_ref[...], w1[...], preferred_element_type=_f32, precision=lax.Precision.HIGHEST) + b1[...], 0.0)
    h = jnp.dot(t, w2[...], preferred_element_type=_f32, precision=lax.Precision.HIGHEST) + b2[...]
    h_ref[...] = h
    p_ref[...] = jnp.dot(h, a0[...], preferred_element_type=_f32, precision=lax.Precision.HIGHEST)
    q_ref[...] = jnp.dot(h, b0[...], preferred_element_type=_f32, precision=lax.Precision.HIGHEST)


def _enc_nodes(x, w1, b1, w2, b2, a0, b0):
    return pl.pallas_call(
        _enc_body,
        grid=(N // BN,),
        in_specs=[
            pl.BlockSpec((BN, x.shape[1]), lambda i: (i, 0)),
            _full(w1.shape), _full(b1.shape), _full(w2.shape), _full(b2.shape),
            _full(a0.shape), _full(b0.shape),
        ],
        out_specs=[_rows((BN, H)), _rows((BN, H)), _rows((BN, H))],
        out_shape=[jax.ShapeDtypeStruct((N, H), _f32)] * 3,
    )(x, w1, b1, w2, b2, a0, b0)


def _ec_body(ea_ref, w1, b1, w2, b2, cs, b1s, ec_ref):
    t = jnp.maximum(jnp.dot(ea_ref[...], w1[...], preferred_element_type=_f32, precision=lax.Precision.HIGHEST) + b1[...], 0.0)
    e = jnp.dot(t, w2[...], preferred_element_type=_f32, precision=lax.Precision.HIGHEST) + b2[...]
    for l in range(N_LAYERS):
        ec_ref[l] = jnp.dot(e, cs[l], preferred_element_type=_f32, precision=lax.Precision.HIGHEST) + b1s[l]


def _ec_all(ea, w1, b1, w2, b2, cs, b1s):
    return pl.pallas_call(
        _ec_body,
        grid=(PAD_E // BE,),
        in_specs=[
            pl.BlockSpec((BE, ea.shape[1]), lambda i: (i, 0)),
            _full(w1.shape), _full(b1.shape), _full(w2.shape), _full(b2.shape),
            pl.BlockSpec(cs.shape, lambda i: (0, 0, 0)),
            pl.BlockSpec(b1s.shape, lambda i: (0, 0, 0)),
        ],
        out_specs=pl.BlockSpec((N_LAYERS, BE, H), lambda i: (0, i, 0)),
        out_shape=jax.ShapeDtypeStruct((N_LAYERS, PAD_E, H), _f32),
    )(ea, w1, b1, w2, b2, cs, b1s)


def _upd_body(h_ref, s_ref, deg_ref, w2e, b2e, v1a, v1b, b1n, v2, b2n, an, bn,
              h_out, p_out, q_out):
    h = h_ref[...]
    agg = jnp.dot(s_ref[...], w2e[...], preferred_element_type=_f32, precision=lax.Precision.HIGHEST) + deg_ref[...] * b2e[...]
    t = jnp.maximum(
        jnp.dot(h, v1a[...], preferred_element_type=_f32, precision=lax.Precision.HIGHEST)
        + jnp.dot(agg, v1b[...], preferred_element_type=_f32, precision=lax.Precision.HIGHEST) + b1n[...], 0.0)
    hn = h + jnp.dot(t, v2[...], preferred_element_type=_f32, precision=lax.Precision.HIGHEST) + b2n[...]
    h_out[...] = hn
    p_out[...] = jnp.dot(hn, an[...], preferred_element_type=_f32, precision=lax.Precision.HIGHEST)
    q_out[...] = jnp.dot(hn, bn[...], preferred_element_type=_f32, precision=lax.Precision.HIGHEST)


def _node_update(h, s, deg, w2e, b2e, v1a, v1b, b1n, v2, b2n, an, bn):
    return pl.pallas_call(
        _upd_body,
        grid=(N // BN,),
        in_specs=[
            _rows((BN, H)), _rows((BN, H)), pl.BlockSpec((BN, 1), lambda i: (i, 0)),
            _full(w2e.shape), _full(b2e.shape), _full(v1a.shape), _full(v1b.shape),
            _full(b1n.shape), _full(v2.shape), _full(b2n.shape),
            _full(an.shape), _full(bn.shape),
        ],
        out_specs=[_rows((BN, H))] * 3,
        out_shape=[jax.ShapeDtypeStruct((N, H), _f32)] * 3,
    )(h, s, deg, w2e, b2e, v1a, v1b, b1n, v2, b2n, an, bn)


def _last_body(h_ref, s_ref, deg_ref, w2e, b2e, v1a, v1b, b1n, v2, b2n,
               d1, db1, d2, db2, d3, db3, fac_ref, out_ref):
    h = h_ref[...]
    agg = jnp.dot(s_ref[...], w2e[...], preferred_element_type=_f32, precision=lax.Precision.HIGHEST) + deg_ref[...] * b2e[...]
    t = jnp.maximum(
        jnp.dot(h, v1a[...], preferred_element_type=_f32, precision=lax.Precision.HIGHEST)
        + jnp.dot(agg, v1b[...], preferred_element_type=_f32, precision=lax.Precision.HIGHEST) + b1n[...], 0.0)
    hn = h + jnp.dot(t, v2[...], preferred_element_type=_f32, precision=lax.Precision.HIGHEST) + b2n[...]
    u = jnp.maximum(jnp.dot(hn, d1[...], preferred_element_type=_f32, precision=lax.Precision.HIGHEST) + db1[...], 0.0)
    u = jnp.maximum(jnp.dot(u, d2[...], preferred_element_type=_f32, precision=lax.Precision.HIGHEST) + db2[...], 0.0)
    raw = jnp.dot(u, d3[...], preferred_element_type=_f32, precision=lax.Precision.HIGHEST) + db3[...]
    out_ref[...] = raw * fac_ref[...]


def _last_update(h, s, deg, w2e, b2e, v1a, v1b, b1n, v2, b2n, dec_ws, fac):
    d1, db1, d2, db2, d3, db3 = dec_ws
    return pl.pallas_call(
        _last_body,
        grid=(N // BN,),
        in_specs=[
            _rows((BN, H)), _rows((BN, H)), pl.BlockSpec((BN, 1), lambda i: (i, 0)),
            _full(w2e.shape), _full(b2e.shape), _full(v1a.shape), _full(v1b.shape),
            _full(b1n.shape), _full(v2.shape), _full(b2n.shape),
            _full(d1.shape), _full(db1.shape), _full(d2.shape), _full(db2.shape),
            _full(d3.shape), _full(db3.shape),
            pl.BlockSpec((BN, 3), lambda i: (i, 0)),
        ],
        out_specs=pl.BlockSpec((BN, 3), lambda i: (i, 0)),
        out_shape=jax.ShapeDtypeStruct((N, 3), _f32),
    )(h, s, deg, w2e, b2e, v1a, v1b, b1n, v2, b2n, d1, db1, d2, db2, d3, db3, fac)


def _build_lists(src, dst, edge_attr):
    c_of = (dst >= NHALF).astype(jnp.int32)
    key = c_of * 2 + (dst & 1)
    pos = jnp.zeros((E,), jnp.int32)
    cnts = []
    for g in range(NLISTS):
        m = (key == g)
        pos = jnp.where(m, jnp.cumsum(m.astype(jnp.int32)) - 1, pos)
        cnts.append(jnp.sum(m.astype(jnp.int32)))
    cnts = jnp.stack(cnts)
    totb = (cnts + BATCH - 1) // BATCH
    subb = (totb + 15) // 16
    padded = subb * 16 * BATCH
    astarts = jnp.cumsum(padded) - padded
    astart_e = jnp.zeros((E,), jnp.int32)
    for g in range(NLISTS):
        astart_e = jnp.where(key == g, astarts[g], astart_e)
    slot = astart_e + pos
    row = (dst - c_of * NHALF) >> 1
    lsrc = jnp.zeros((PAD_E,), jnp.int32).at[slot].set(src)
    ldstg = jnp.zeros((PAD_E,), jnp.int32).at[slot].set(dst)
    ldstl = jnp.full((PAD_E,), DUMP, jnp.int32).at[slot].set(row)
    ea_s = jnp.zeros((PAD_E, edge_attr.shape[1]), _f32).at[slot].set(edge_attr)
    def splat(v):
        return jnp.broadcast_to(v.astype(jnp.int32)[:, None], (NLISTS, 16)).reshape(-1)
    return lsrc, ldstg, ldstl, splat(subb), splat(totb), splat(astarts), ea_s


def _sc_mesh():
    return plsc.VectorSubcoreMesh(core_axis_name="c", subcore_axis_name="s")


def _zero_buf(buf, rows, width):
    zrow = jnp.zeros((16,), _f32)

    def zb(j, _):
        for kk in range(width // 16):
            buf[j, pl.ds(kk * 16, 16)] = zrow
        return 0
    lax.fori_loop(0, rows, zb, 0)


def _zero_acc(acc, zb, s):
    base = s * ZCH

    def za(j, _):
        pltpu.sync_copy(zb, acc.at[pl.ds(base + j * BATCH, BATCH)])
        return 0
    lax.fori_loop(0, ZCH // BATCH, za, 0)
    pltpu.sync_copy(zb.at[pl.ds(0, ZCH % BATCH)],
                    acc.at[pl.ds(base + (ZCH // BATCH) * BATCH, ZCH % BATCH)])


def _drain_acc(acc, out_h, c, s):
    row0 = s * DCH
    gbase = c * 12504 + row0

    def dr(j, _):
        pltpu.sync_copy(acc.at[pl.ds(row0 + j * 128, 128)],
                        out_h.at[pl.ds(gbase + j * 128, 128)])
        return 0
    lax.fori_loop(0, 5, dr, 0)

    @pl.when(s < 15)
    def _t1():
        pltpu.sync_copy(acc.at[pl.ds(row0 + 640, 128)],
                        out_h.at[pl.ds(gbase + 640, 128)])
        pltpu.sync_copy(acc.at[pl.ds(row0 + 768, 16)],
                        out_h.at[pl.ds(gbase + 768, 16)])

    @pl.when(s == 15)
    def _t2():
        pltpu.sync_copy(acc.at[pl.ds(row0 + 640, 104)],
                        out_h.at[pl.ds(gbase + 640, 104)])


def _sc_layer(PQ, ecl, lsrc, ldstg, ldstl, subb, totb, starts):
    @functools.partial(
        pl.kernel, mesh=_sc_mesh(),
        out_type=jax.ShapeDtypeStruct((OUT_ROWS, 2 * H), _f32),
        scratch_types=[
            pltpu.VMEM_SHARED((ACC_ROWS, 2 * H), _f32),
            pltpu.VMEM((BATCH,), jnp.int32),
            pltpu.VMEM((BATCH,), jnp.int32),
            pltpu.VMEM((BATCH,), jnp.int32),
            pltpu.VMEM((BATCH, 2 * H), _f32),
            pltpu.VMEM((BATCH, 2 * H), _f32),
            pltpu.VMEM((BATCH, H), _f32),
            pltpu.VMEM((BATCH, 2 * H), _f32),
            pltpu.VMEM((16,), jnp.int32),
            pltpu.VMEM((16,), jnp.int32),
            pltpu.VMEM((16,), jnp.int32),
            pltpu.SemaphoreType.DMA,
            pltpu.SemaphoreType.DMA,
            pltpu.SemaphoreType.DMA,
        ],
    )
    def k(PQ_h, ec_h, lsrc_h, ldstg_h, ldstl_h, sb_h, tb_h, st_h, out_h,
          acc, srcv, dstgv, dstlv, bufS, bufD, bufE, bufZ, cntv, stv, tbv,
          sem1, sem2, sem3):
        c = lax.axis_index("c")
        s = lax.axis_index("s")
        _zero_buf(bufZ, BATCH, 2 * H)
        _zero_acc(acc, bufZ, s)
        plsc.subcore_barrier()

        def do_list(par):
            g = c * 2 + par
            pltpu.sync_copy(sb_h.at[pl.ds(g * 16, 16)], cntv)
            pltpu.sync_copy(tb_h.at[pl.ds(g * 16, 16)], tbv)
            pltpu.sync_copy(st_h.at[pl.ds(g * 16, 16)], stv)
            sb = cntv[pl.ds(0, 16)][0]
            tb = tbv[pl.ds(0, 16)][0]
            nb = jnp.minimum(sb, jnp.maximum(tb - s * sb, 0))
            st = pl.multiple_of(stv[pl.ds(0, 16)][0] + s * sb * BATCH, BATCH)

            def body(i, _):
                off = pl.multiple_of(st + i * BATCH, BATCH)
                pltpu.sync_copy(lsrc_h.at[pl.ds(off, BATCH)], srcv)
                pltpu.sync_copy(ldstg_h.at[pl.ds(off, BATCH)], dstgv)
                pltpu.sync_copy(ldstl_h.at[pl.ds(off, BATCH)], dstlv)
                cp1 = pltpu.async_copy(PQ_h.at[srcv], bufS, sem1)
                cp2 = pltpu.async_copy(PQ_h.at[dstgv], bufD, sem2)
                cp3 = pltpu.async_copy(ec_h.at[pl.ds(off, BATCH)], bufE, sem3)
                cp1.wait()
                cp2.wait()
                cp3.wait()

                def rowf(j, _):
                    for kk in range(4):
                        bufZ[j, pl.ds(par * H + kk * 16, 16)] = jnp.maximum(
                            bufS[j, pl.ds(kk * 16, 16)]
                            + bufD[j, pl.ds(H + kk * 16, 16)]
                            + bufE[j, pl.ds(kk * 16, 16)], 0.0)
                    return 0
                lax.fori_loop(0, BATCH, rowf, 0)
                pltpu.sync_copy(bufZ, acc.at[dstlv], add=True)
                return 0
            lax.fori_loop(0, nb, body, 0)

        for par in (0, 1):
            zrow = jnp.zeros((16,), _f32)

            def zhalf(j, _):
                for kk in range(4):
                    bufZ[j, pl.ds((1 - par) * H + kk * 16, 16)] = zrow
                return 0
            lax.fori_loop(0, BATCH, zhalf, 0)
            do_list(par)
        plsc.subcore_barrier()
        _drain_acc(acc, out_h, c, s)

    return k(PQ, ecl, lsrc, ldstg, ldstl, subb, totb, starts)


def _unpack_s(out):
    return jnp.concatenate([out[0:12500], out[12504:25004]], axis=0).reshape(N, H)


def kernel(x, edge_index, edge_attr, u_c, theta_c, bc_disp, bc_rot, params):
    src = edge_index[0]
    dst = edge_index[1]

    def r2(b):
        return b.reshape(1, -1)

    ne = params['node_enc']
    ee = params['edge_enc']
    mp = params['mp']
    dec = params['dec']

    As = [lp['edge'][0][0][0:H] for lp in mp]
    Bs = [lp['edge'][0][0][H:2 * H] for lp in mp]
    Cs = jnp.stack([lp['edge'][0][0][2 * H:3 * H] for lp in mp])
    b1s = jnp.stack([lp['edge'][0][1].reshape(1, H) for lp in mp])
    V1as = [lp['node'][0][0][0:H] for lp in mp]
    V1bs = [lp['node'][0][0][H:2 * H] for lp in mp]

    lsrc, ldstg, ldstl, subb, totb, starts, ea_s = _build_lists(src, dst, edge_attr)

    h, P, Q = _enc_nodes(x, ne[0][0], r2(ne[0][1]), ne[1][0], r2(ne[1][1]),
                         As[0], Bs[0])
    ec = _ec_all(ea_s, ee[0][0], r2(ee[0][1]), ee[1][0], r2(ee[1][1]),
                 Cs, b1s)

    deg = _unpack_s(_sc_layer(jnp.zeros((N, 2 * H), _f32),
                              jnp.ones((PAD_E, H), _f32),
                              lsrc, ldstg, ldstl, subb, totb, starts))[:, 0:1]

    fac = jnp.concatenate([
        u_c.reshape(N, 1) * (1.0 - bc_disp),
        u_c.reshape(N, 1) * (1.0 - bc_disp),
        theta_c.reshape(N, 1) * (1.0 - bc_rot)], axis=1)

    for l in range(N_LAYERS):
        lp = mp[l]
        s = _unpack_s(_sc_layer(jnp.concatenate([P, Q], axis=1), ec[l],
                                lsrc, ldstg, ldstl, subb, totb, starts))
        w2e, b2e = lp['edge'][1][0], r2(lp['edge'][1][1])
        b1n = r2(lp['node'][0][1])
        v2, b2n = lp['node'][1][0], r2(lp['node'][1][1])
        if l < N_LAYERS - 1:
            h, P, Q = _node_update(h, s, deg, w2e, b2e, V1as[l], V1bs[l], b1n,
                                   v2, b2n, As[l + 1], Bs[l + 1])
        else:
            dec_ws = (dec[0][0], r2(dec[0][1]), dec[1][0], r2(dec[1][1]),
                      dec[2][0], r2(dec[2][1]))
            out = _last_update(h, s, deg, w2e, b2e, V1as[l], V1bs[l], b1n,
                               v2, b2n, dec_ws, fac)
    return out


# speedup vs baseline: 3.2405x; 1.0203x over previous
import functools
import jax
import jax.numpy as jnp
from jax import lax
from jax.experimental import pallas as pl
from jax.experimental.pallas import tpu as pltpu
from jax.experimental.pallas import tpu_sc as plsc

N = 50000
E = 800000
H = 64
N_LAYERS = 6
BN = 1000
BE = 1024

NHALF = 25000
BATCH = 56
NLISTS = 4
PAD_E = 804864
ACC_ROWS = 12672
DUMP = 12544
ZCH = 792
DCH = 784
OUT_ROWS = 25008

_f32 = jnp.float32


def _full(shape):
    return pl.BlockSpec(shape, lambda *_: tuple(0 for _ in shape))


def _rows(shape):
    return pl.BlockSpec(shape, lambda i: (0,) * (len(shape) - 2) + (i, 0))


def _enc_body(x_ref, w1, b1, w2, b2, a0, b0, h_ref, p_ref, q_ref):
    t = jnp.maximum(jnp.dot(x_ref[...], w1[...], preferred_element_type=_f32, precision=lax.Precision.HIGHEST) + b1[...], 0.0)
    h = jnp.dot(t, w2[...], preferred_element_type=_f32, precision=lax.Precision.HIGHEST) + b2[...]
    h_ref[...] = h
    p_ref[...] = jnp.dot(h, a0[...], preferred_element_type=_f32, precision=lax.Precision.HIGHEST)
    q_ref[...] = jnp.dot(h, b0[...], preferred_element_type=_f32, precision=lax.Precision.HIGHEST)


def _enc_nodes(x, w1, b1, w2, b2, a0, b0):
    return pl.pallas_call(
        _enc_body,
        grid=(N // BN,),
        in_specs=[
            pl.BlockSpec((BN, x.shape[1]), lambda i: (i, 0)),
            _full(w1.shape), _full(b1.shape), _full(w2.shape), _full(b2.shape),
            _full(a0.shape), _full(b0.shape),
        ],
        out_specs=[_rows((BN, H)), _rows((BN, H)), _rows((BN, H))],
        out_shape=[jax.ShapeDtypeStruct((N, H), _f32)] * 3,
    )(x, w1, b1, w2, b2, a0, b0)


def _ec_body(ea_ref, w1, b1, w2, b2, cs, b1s, ec_ref):
    t = jnp.maximum(jnp.dot(ea_ref[...], w1[...], preferred_element_type=_f32, precision=lax.Precision.HIGHEST) + b1[...], 0.0)
    e = jnp.dot(t, w2[...], preferred_element_type=_f32, precision=lax.Precision.HIGHEST) + b2[...]
    for l in range(N_LAYERS):
        ec_ref[l] = jnp.dot(e, cs[l], preferred_element_type=_f32, precision=lax.Precision.HIGHEST) + b1s[l]


def _ec_all(ea, w1, b1, w2, b2, cs, b1s):
    return pl.pallas_call(
        _ec_body,
        grid=(PAD_E // BE,),
        in_specs=[
            pl.BlockSpec((BE, ea.shape[1]), lambda i: (i, 0)),
            _full(w1.shape), _full(b1.shape), _full(w2.shape), _full(b2.shape),
            pl.BlockSpec(cs.shape, lambda i: (0, 0, 0)),
            pl.BlockSpec(b1s.shape, lambda i: (0, 0, 0)),
        ],
        out_specs=pl.BlockSpec((N_LAYERS, BE, H), lambda i: (0, i, 0)),
        out_shape=jax.ShapeDtypeStruct((N_LAYERS, PAD_E, H), _f32),
    )(ea, w1, b1, w2, b2, cs, b1s)


def _upd_body(h_ref, s_ref, deg_ref, w2e, b2e, v1a, v1b, b1n, v2, b2n, an, bn,
              h_out, p_out, q_out):
    h = h_ref[...]
    agg = jnp.dot(s_ref[...], w2e[...], preferred_element_type=_f32, precision=lax.Precision.HIGHEST) + deg_ref[...] * b2e[...]
    t = jnp.maximum(
        jnp.dot(h, v1a[...], preferred_element_type=_f32, precision=lax.Precision.HIGHEST)
        + jnp.dot(agg, v1b[...], preferred_element_type=_f32, precision=lax.Precision.HIGHEST) + b1n[...], 0.0)
    hn = h + jnp.dot(t, v2[...], preferred_element_type=_f32, precision=lax.Precision.HIGHEST) + b2n[...]
    h_out[...] = hn
    p_out[...] = jnp.dot(hn, an[...], preferred_element_type=_f32, precision=lax.Precision.HIGHEST)
    q_out[...] = jnp.dot(hn, bn[...], preferred_element_type=_f32, precision=lax.Precision.HIGHEST)


def _node_update(h, s, deg, w2e, b2e, v1a, v1b, b1n, v2, b2n, an, bn):
    return pl.pallas_call(
        _upd_body,
        grid=(N // BN,),
        in_specs=[
            _rows((BN, H)), _rows((BN, H)), pl.BlockSpec((BN, 1), lambda i: (i, 0)),
            _full(w2e.shape), _full(b2e.shape), _full(v1a.shape), _full(v1b.shape),
            _full(b1n.shape), _full(v2.shape), _full(b2n.shape),
            _full(an.shape), _full(bn.shape),
        ],
        out_specs=[_rows((BN, H))] * 3,
        out_shape=[jax.ShapeDtypeStruct((N, H), _f32)] * 3,
    )(h, s, deg, w2e, b2e, v1a, v1b, b1n, v2, b2n, an, bn)


def _last_body(h_ref, s_ref, deg_ref, w2e, b2e, v1a, v1b, b1n, v2, b2n,
               d1, db1, d2, db2, d3, db3, fac_ref, out_ref):
    h = h_ref[...]
    agg = jnp.dot(s_ref[...], w2e[...], preferred_element_type=_f32, precision=lax.Precision.HIGHEST) + deg_ref[...] * b2e[...]
    t = jnp.maximum(
        jnp.dot(h, v1a[...], preferred_element_type=_f32, precision=lax.Precision.HIGHEST)
        + jnp.dot(agg, v1b[...], preferred_element_type=_f32, precision=lax.Precision.HIGHEST) + b1n[...], 0.0)
    hn = h + jnp.dot(t, v2[...], preferred_element_type=_f32, precision=lax.Precision.HIGHEST) + b2n[...]
    u = jnp.maximum(jnp.dot(hn, d1[...], preferred_element_type=_f32, precision=lax.Precision.HIGHEST) + db1[...], 0.0)
    u = jnp.maximum(jnp.dot(u, d2[...], preferred_element_type=_f32, precision=lax.Precision.HIGHEST) + db2[...], 0.0)
    raw = jnp.dot(u, d3[...], preferred_element_type=_f32, precision=lax.Precision.HIGHEST) + db3[...]
    out_ref[...] = raw * fac_ref[...]


def _last_update(h, s, deg, w2e, b2e, v1a, v1b, b1n, v2, b2n, dec_ws, fac):
    d1, db1, d2, db2, d3, db3 = dec_ws
    return pl.pallas_call(
        _last_body,
        grid=(N // BN,),
        in_specs=[
            _rows((BN, H)), _rows((BN, H)), pl.BlockSpec((BN, 1), lambda i: (i, 0)),
            _full(w2e.shape), _full(b2e.shape), _full(v1a.shape), _full(v1b.shape),
            _full(b1n.shape), _full(v2.shape), _full(b2n.shape),
            _full(d1.shape), _full(db1.shape), _full(d2.shape), _full(db2.shape),
            _full(d3.shape), _full(db3.shape),
            pl.BlockSpec((BN, 3), lambda i: (i, 0)),
        ],
        out_specs=pl.BlockSpec((BN, 3), lambda i: (i, 0)),
        out_shape=jax.ShapeDtypeStruct((N, 3), _f32),
    )(h, s, deg, w2e, b2e, v1a, v1b, b1n, v2, b2n, d1, db1, d2, db2, d3, db3, fac)


def _build_lists(src, dst, edge_attr):
    c_of = (dst >= NHALF).astype(jnp.int32)
    key = c_of * 2 + (dst & 1)
    pos = jnp.zeros((E,), jnp.int32)
    cnts = []
    for g in range(NLISTS):
        m = (key == g)
        pos = jnp.where(m, jnp.cumsum(m.astype(jnp.int32)) - 1, pos)
        cnts.append(jnp.sum(m.astype(jnp.int32)))
    cnts = jnp.stack(cnts)
    totb = (cnts + BATCH - 1) // BATCH
    subb = (totb + 15) // 16
    padded = subb * 16 * BATCH
    astarts = jnp.cumsum(padded) - padded
    astart_e = jnp.zeros((E,), jnp.int32)
    for g in range(NLISTS):
        astart_e = jnp.where(key == g, astarts[g], astart_e)
    slot = astart_e + pos
    row = (dst - c_of * NHALF) >> 1
    lsrc = jnp.zeros((PAD_E,), jnp.int32).at[slot].set(src)
    ldstg = jnp.zeros((PAD_E,), jnp.int32).at[slot].set(dst)
    ldstl = jnp.full((PAD_E,), DUMP, jnp.int32).at[slot].set(row)
    ea_s = jnp.zeros((PAD_E, edge_attr.shape[1]), _f32).at[slot].set(edge_attr)
    def splat(v):
        return jnp.broadcast_to(v.astype(jnp.int32)[:, None], (NLISTS, 16)).reshape(-1)
    return lsrc, ldstg, ldstl, splat(subb), splat(totb), splat(astarts), ea_s


def _sc_mesh():
    return plsc.VectorSubcoreMesh(core_axis_name="c", subcore_axis_name="s")


def _zero_buf(buf, rows, width):
    zrow = jnp.zeros((16,), _f32)

    def zb(j, _):
        for kk in range(width // 16):
            buf[j, pl.ds(kk * 16, 16)] = zrow
        return 0
    lax.fori_loop(0, rows, zb, 0)


def _zero_acc(acc, zb, s):
    base = s * ZCH

    def za(j, _):
        pltpu.sync_copy(zb, acc.at[pl.ds(base + j * BATCH, BATCH)])
        return 0
    lax.fori_loop(0, ZCH // BATCH, za, 0)
    pltpu.sync_copy(zb.at[pl.ds(0, ZCH % BATCH)],
                    acc.at[pl.ds(base + (ZCH // BATCH) * BATCH, ZCH % BATCH)])


def _drain_acc(acc, out_h, c, s):
    row0 = s * DCH
    gbase = c * 12504 + row0

    def dr(j, _):
        pltpu.sync_copy(acc.at[pl.ds(row0 + j * 128, 128)],
                        out_h.at[pl.ds(gbase + j * 128, 128)])
        return 0
    lax.fori_loop(0, 5, dr, 0)

    @pl.when(s < 15)
    def _t1():
        pltpu.sync_copy(acc.at[pl.ds(row0 + 640, 128)],
                        out_h.at[pl.ds(gbase + 640, 128)])
        pltpu.sync_copy(acc.at[pl.ds(row0 + 768, 16)],
                        out_h.at[pl.ds(gbase + 768, 16)])

    @pl.when(s == 15)
    def _t2():
        pltpu.sync_copy(acc.at[pl.ds(row0 + 640, 104)],
                        out_h.at[pl.ds(gbase + 640, 104)])


def _sc_layer(PQ, ecl, lsrc, ldstg, ldstl, subb, totb, starts):
    @functools.partial(
        pl.kernel, mesh=_sc_mesh(),
        out_type=jax.ShapeDtypeStruct((OUT_ROWS, 2 * H), _f32),
        scratch_types=[
            pltpu.VMEM_SHARED((ACC_ROWS, 2 * H), _f32),
            pltpu.VMEM((BATCH,), jnp.int32),
            pltpu.VMEM((BATCH,), jnp.int32),
            pltpu.VMEM((BATCH,), jnp.int32),
            pltpu.VMEM((BATCH, 2 * H), _f32),
            pltpu.VMEM((BATCH, 2 * H), _f32),
            pltpu.VMEM((BATCH, H), _f32),
            pltpu.VMEM((BATCH, 2 * H), _f32),
            pltpu.VMEM((16,), jnp.int32),
            pltpu.VMEM((16,), jnp.int32),
            pltpu.VMEM((16,), jnp.int32),
            pltpu.SemaphoreType.DMA,
            pltpu.SemaphoreType.DMA,
            pltpu.SemaphoreType.DMA,
        ],
    )
    def k(PQ_h, ec_h, lsrc_h, ldstg_h, ldstl_h, sb_h, tb_h, st_h, out_h,
          acc, srcv, dstgv, dstlv, bufS, bufD, bufE, bufZ, cntv, stv, tbv,
          sem1, sem2, sem3):
        c = lax.axis_index("c")
        s = lax.axis_index("s")
        _zero_buf(bufZ, BATCH, 2 * H)
        _zero_acc(acc, bufZ, s)
        plsc.subcore_barrier()

        def do_list(par):
            g = c * 2 + par
            pltpu.sync_copy(sb_h.at[pl.ds(g * 16, 16)], cntv)
            pltpu.sync_copy(tb_h.at[pl.ds(g * 16, 16)], tbv)
            pltpu.sync_copy(st_h.at[pl.ds(g * 16, 16)], stv)
            sb = cntv[pl.ds(0, 16)][0]
            tb = tbv[pl.ds(0, 16)][0]
            nb = jnp.minimum(sb, jnp.maximum(tb - s * sb, 0))
            st = pl.multiple_of(stv[pl.ds(0, 16)][0] + s * sb * BATCH, BATCH)

            def body(i, _):
                off = pl.multiple_of(st + i * BATCH, BATCH)
                pltpu.sync_copy(lsrc_h.at[pl.ds(off, BATCH)], srcv)
                pltpu.sync_copy(ldstg_h.at[pl.ds(off, BATCH)], dstgv)
                pltpu.sync_copy(ldstl_h.at[pl.ds(off, BATCH)], dstlv)
                cp1 = pltpu.async_copy(PQ_h.at[srcv], bufS, sem1)
                cp2 = pltpu.async_copy(PQ_h.at[dstgv], bufD, sem2)
                cp3 = pltpu.async_copy(ec_h.at[pl.ds(off, BATCH)], bufE, sem3)
                cp1.wait()
                cp2.wait()
                cp3.wait()

                def rowf(j, _):
                    for kk in range(4):
                        bufZ[j, pl.ds(par * H + kk * 16, 16)] = jnp.maximum(
                            bufS[j, pl.ds(kk * 16, 16)]
                            + bufD[j, pl.ds(H + kk * 16, 16)]
                            + bufE[j, pl.ds(kk * 16, 16)], 0.0)
                    return 0
                lax.fori_loop(0, BATCH, rowf, 0)
                pltpu.sync_copy(bufZ, acc.at[dstlv], add=True)
                return 0
            lax.fori_loop(0, nb, body, 0)

        for par in (0, 1):
            zrow = jnp.zeros((16,), _f32)

            def zhalf(j, _):
                for kk in range(4):
                    bufZ[j, pl.ds((1 - par) * H + kk * 16, 16)] = zrow
                return 0
            lax.fori_loop(0, BATCH, zhalf, 0)
            do_list(par)
        plsc.subcore_barrier()
        _drain_acc(acc, out_h, c, s)

    return k(PQ, ecl, lsrc, ldstg, ldstl, subb, totb, starts)


def _unpack_s(out):
    return jnp.concatenate([out[0:12500], out[12504:25004]], axis=0).reshape(N, H)


def kernel(x, edge_index, edge_attr, u_c, theta_c, bc_disp, bc_rot, params):
    src = edge_index[0]
    dst = edge_index[1]

    def r2(b):
        return b.reshape(1, -1)

    ne = params['node_enc']
    ee = params['edge_enc']
    mp = params['mp']
    dec = params['dec']

    As = [lp['edge'][0][0][0:H] for lp in mp]
    Bs = [lp['edge'][0][0][H:2 * H] for lp in mp]
    Cs = jnp.stack([lp['edge'][0][0][2 * H:3 * H] for lp in mp])
    b1s = jnp.stack([lp['edge'][0][1].reshape(1, H) for lp in mp])
    V1as = [lp['node'][0][0][0:H] for lp in mp]
    V1bs = [lp['node'][0][0][H:2 * H] for lp in mp]

    lsrc, ldstg, ldstl, subb, totb, starts, ea_s = _build_lists(src, dst, edge_attr)

    h, P, Q = _enc_nodes(x, ne[0][0], r2(ne[0][1]), ne[1][0], r2(ne[1][1]),
                         As[0], Bs[0])
    ec = _ec_all(ea_s, ee[0][0], r2(ee[0][1]), ee[1][0], r2(ee[1][1]),
                 Cs, b1s)

    deg = _unpack_s(_sc_layer(jnp.zeros((N, 2 * H), _f32),
                              jnp.ones((PAD_E, H), _f32),
                              lsrc, ldstg, ldstl, subb, totb, starts))[:, 0:1]

    fac = jnp.concatenate([
        u_c.reshape(N, 1) * (1.0 - bc_disp),
        u_c.reshape(N, 1) * (1.0 - bc_disp),
        theta_c.reshape(N, 1) * (1.0 - bc_rot)], axis=1)

    for l in range(N_LAYERS):
        lp = mp[l]
        s = _unpack_s(_sc_layer(jnp.concatenate([P, Q], axis=1), ec[l],
                                lsrc, ldstg, ldstl, subb, totb, starts))
        w2e, b2e = lp['edge'][1][0], r2(lp['edge'][1][1])
        b1n = r2(lp['node'][0][1])
        v2, b2n = lp['node'][1][0], r2(lp['node'][1][1])
        if l < N_LAYERS - 1:
            h, P, Q = _node_update(h, s, deg, w2e, b2e, V1as[l], V1bs[l], b1n,
                                   v2, b2n, As[l + 1], Bs[l + 1])
        else:
            dec_ws = (dec[0][0], r2(dec[0][1]), dec[1][0], r2(dec[1][1]),
                      dec[2][0], r2(dec[2][1]))
            out = _last_update(h, s, deg, w2e, b2e, V1as[l], V1bs[l], b1n,
                               v2, b2n, dec_ws, fac)
    return out
